# baseline (speedup 1.0000x reference)
import functools

import jax
import jax.numpy as jnp
from jax.experimental import pallas as pl
from jax.experimental.pallas import tpu as pltpu

N = 2048
D_IN = 512
B_HID = 4096
CODE = 64
NUM_GRIDS = 8
GRID_MIN, GRID_MAX = -2.0, 2.0
BM = 256
NSTEPS = N // BM

_BF = jnp.bfloat16


def _dot(a, b):
    return jnp.dot(a, b, preferred_element_type=jnp.float32)


def _body(G_ref, x_ref, W1_ref, b1_ref, W2_ref, b2_ref, lnw_ref, lnb_ref,
          W3p_ref, b3_ref, code_ref, out_ref,
          xb_ref, W1b_ref, W2b_ref, W3b_ref, Gb_ref, u_ref):
    s = pl.program_id(0)

    @pl.when(s == 0)
    def _init():
        xb_ref[...] = x_ref[...].astype(_BF)
        W1b_ref[...] = W1_ref[...].astype(_BF)
        W2b_ref[...] = W2_ref[...].astype(_BF)
        W3b_ref[...] = W3p_ref[...].astype(_BF)

    @pl.when(s < NSTEPS)
    def _encode():
        Gb = G_ref[...].astype(_BF)
        Gb_ref[pl.ds(s * BM, BM), :] = Gb
        t = _dot(Gb, xb_ref[...])
        h = jnp.maximum(_dot(t.astype(_BF), W1b_ref[...]) + b1_ref[...], 0.0)
        u_ref[pl.ds(s * BM, BM), :] = _dot(h.astype(_BF), W2b_ref[...]).astype(_BF)

    @pl.when(s >= NSTEPS)
    def _decode():
        i = s - NSTEPS
        Gt = Gb_ref[pl.ds(i * BM, BM), :]
        feat = _dot(Gt, u_ref[...]) + b2_ref[...]
        code = jnp.tanh(10.0 * feat)
        code_ref[...] = code
        mu = jnp.mean(code, axis=-1, keepdims=True)
        var = jnp.mean((code - mu) ** 2, axis=-1, keepdims=True)
        y = (code - mu) * jax.lax.rsqrt(var + 1e-5) * lnw_ref[...] + lnb_ref[...]
        denom = (GRID_MAX - GRID_MIN) / (NUM_GRIDS - 1)
        rbf_blocks = []
        for g in range(NUM_GRIDS):
            gval = GRID_MIN + denom * g
            rbf_blocks.append(jnp.exp(-(((y - gval) / denom) ** 2)))
        rbf = jnp.concatenate(rbf_blocks, axis=-1)
        out = _dot(rbf.astype(_BF), W3b_ref[...]) + b3_ref[...]
        out_ref[...] = jnp.maximum(out, 0.0)


def kernel(x, G, W1, b1, W2, b2, ln_w, ln_b, W3, b3):
    b1r = b1.reshape(1, B_HID)
    b2r = b2.reshape(1, CODE)
    lnwr = ln_w.reshape(1, CODE)
    lnbr = ln_b.reshape(1, CODE)
    b3r = b3.reshape(1, 2 * D_IN)
    W3p = W3.reshape(CODE, NUM_GRIDS, 2 * D_IN).transpose(1, 0, 2) \
             .reshape(NUM_GRIDS * CODE, 2 * D_IN)

    last = NSTEPS - 1
    code, feat_out = pl.pallas_call(
        _body,
        grid=(2 * NSTEPS,),
        in_specs=[
            pl.BlockSpec((BM, N), lambda s: (jnp.minimum(s, last), 0)),
            pl.BlockSpec((N, D_IN), lambda s: (0, 0)),
            pl.BlockSpec((D_IN, B_HID), lambda s: (0, 0)),
            pl.BlockSpec((1, B_HID), lambda s: (0, 0)),
            pl.BlockSpec((B_HID, CODE), lambda s: (0, 0)),
            pl.BlockSpec((1, CODE), lambda s: (0, 0)),
            pl.BlockSpec((1, CODE), lambda s: (0, 0)),
            pl.BlockSpec((1, CODE), lambda s: (0, 0)),
            pl.BlockSpec((NUM_GRIDS * CODE, 2 * D_IN), lambda s: (0, 0)),
            pl.BlockSpec((1, 2 * D_IN), lambda s: (0, 0)),
        ],
        out_specs=[
            pl.BlockSpec((BM, CODE), lambda s: (jnp.maximum(s - NSTEPS, 0), 0)),
            pl.BlockSpec((BM, 2 * D_IN), lambda s: (jnp.maximum(s - NSTEPS, 0), 0)),
        ],
        out_shape=[
            jax.ShapeDtypeStruct((N, CODE), jnp.float32),
            jax.ShapeDtypeStruct((N, 2 * D_IN), jnp.float32),
        ],
        scratch_shapes=[
            pltpu.VMEM((N, D_IN), _BF),
            pltpu.VMEM((D_IN, B_HID), _BF),
            pltpu.VMEM((B_HID, CODE), _BF),
            pltpu.VMEM((NUM_GRIDS * CODE, 2 * D_IN), _BF),
            pltpu.VMEM((N, N), _BF),
            pltpu.VMEM((N, CODE), _BF),
        ],
    )(G, x, W1, b1r, W2, b2r, lnwr, lnbr, W3p, b3r)

    return (code, feat_out)
